# baseline (speedup 1.0000x reference)
import jax
import jax.numpy as jnp
from jax.experimental import pallas as pl
from jax.experimental.pallas import tpu as pltpu

_EPS = 1e-05
_GROUP = 64


def _kblk(d, limit=4096):
    best = 128
    for m in range(128, limit + 1, 128):
        if d % m == 0:
            best = m
    return best


def _pass1_body(x_ref, w_ref, logits_ref, rsum_ref, rsumsq_ref):
    k = pl.program_id(0)

    @pl.when(k == 0)
    def _():
        logits_ref[...] = jnp.zeros_like(logits_ref)
        rsum_ref[...] = jnp.zeros_like(rsum_ref)
        rsumsq_ref[...] = jnp.zeros_like(rsumsq_ref)

    xb = x_ref[...]
    wb = w_ref[...]
    logits_ref[...] += jax.lax.dot_general(
        xb.astype(jnp.bfloat16), wb.astype(jnp.bfloat16),
        (((1,), (1,)), ((), ())), preferred_element_type=jnp.float32)
    rsum_ref[...] += jnp.sum(xb, axis=1, keepdims=True)
    rsumsq_ref[...] += jnp.sum(xb * xb, axis=1, keepdims=True)


def _stats_body(logits_ref, b_ref, rsum_ref, rsumsq_ref, wrow_ref, brow_ref,
                d_ref, scale_ref, shift_ref):
    logits = logits_ref[...] + b_ref[...]
    d = d_ref[0]
    rows, group = logits.shape
    iota = jax.lax.broadcasted_iota(jnp.int32, (rows, group), 1)
    rowmax = jnp.max(logits, axis=1, keepdims=True)
    s = jnp.min(jnp.where(logits == rowmax, iota, group), axis=1, keepdims=True)
    onehot = (s == iota).astype(jnp.float32)
    cnt = jnp.sum(onehot, axis=0, keepdims=True)
    g_sum = jnp.sum(onehot * rsum_ref[...], axis=0, keepdims=True)
    g_sumsq = jnp.sum(onehot * rsumsq_ref[...], axis=0, keepdims=True)
    n_el = cnt * d
    mean = g_sum / jnp.maximum(n_el, 1.0)
    var = (g_sumsq - n_el * mean * mean) / jnp.maximum(n_el - 1.0, 1.0)
    m_r = jnp.sum(onehot * mean, axis=1, keepdims=True)
    v_r = jnp.sum(onehot * var, axis=1, keepdims=True)
    inv = wrow_ref[...] * jax.lax.rsqrt(v_r + _EPS)
    scale_ref[...] = inv
    shift_ref[...] = brow_ref[...] - m_r * inv


def _pass3_body(x_ref, scale_ref, shift_ref, out_ref):
    out_ref[...] = x_ref[...] * scale_ref[...] + shift_ref[...]


def kernel(x, fc_w, fc_b, weight, bias):
    n, c, h, w = x.shape
    d = h * w
    rows = n * c
    x_ = x.reshape(rows, d)
    wrow = jnp.broadcast_to(weight.reshape(1, c), (n, c)).reshape(rows, 1)
    brow = jnp.broadcast_to(bias.reshape(1, c), (n, c)).reshape(rows, 1)

    kblk = _kblk(d)
    ksteps = d // kblk

    logits, rsum, rsumsq = pl.pallas_call(
        _pass1_body,
        grid=(ksteps,),
        in_specs=[
            pl.BlockSpec((rows, kblk), lambda k: (0, k)),
            pl.BlockSpec((_GROUP, kblk), lambda k: (0, k)),
        ],
        out_specs=[
            pl.BlockSpec((rows, _GROUP), lambda k: (0, 0)),
            pl.BlockSpec((rows, 1), lambda k: (0, 0)),
            pl.BlockSpec((rows, 1), lambda k: (0, 0)),
        ],
        out_shape=[
            jax.ShapeDtypeStruct((rows, _GROUP), jnp.float32),
            jax.ShapeDtypeStruct((rows, 1), jnp.float32),
            jax.ShapeDtypeStruct((rows, 1), jnp.float32),
        ],
        compiler_params=pltpu.CompilerParams(
            dimension_semantics=("arbitrary",)),
    )(x_, fc_w)

    dval = jnp.full((1,), float(d), dtype=jnp.float32)
    scale, shift = pl.pallas_call(
        _stats_body,
        in_specs=[
            pl.BlockSpec((rows, _GROUP), lambda: (0, 0)),
            pl.BlockSpec((1, _GROUP), lambda: (0, 0)),
            pl.BlockSpec((rows, 1), lambda: (0, 0)),
            pl.BlockSpec((rows, 1), lambda: (0, 0)),
            pl.BlockSpec((rows, 1), lambda: (0, 0)),
            pl.BlockSpec((rows, 1), lambda: (0, 0)),
            pl.BlockSpec(memory_space=pltpu.SMEM),
        ],
        out_specs=[
            pl.BlockSpec((rows, 1), lambda: (0, 0)),
            pl.BlockSpec((rows, 1), lambda: (0, 0)),
        ],
        out_shape=[
            jax.ShapeDtypeStruct((rows, 1), jnp.float32),
            jax.ShapeDtypeStruct((rows, 1), jnp.float32),
        ],
    )(logits, fc_b.reshape(1, _GROUP), rsum, rsumsq, wrow, brow, dval)

    out = pl.pallas_call(
        _pass3_body,
        grid=(ksteps,),
        in_specs=[
            pl.BlockSpec((rows, kblk), lambda k: (0, k)),
            pl.BlockSpec((rows, 1), lambda k: (0, 0)),
            pl.BlockSpec((rows, 1), lambda k: (0, 0)),
        ],
        out_specs=pl.BlockSpec((rows, kblk), lambda k: (0, k)),
        out_shape=jax.ShapeDtypeStruct((rows, d), jnp.float32),
        compiler_params=pltpu.CompilerParams(
            dimension_semantics=("arbitrary",)),
    )(x_, scale, shift)

    return out.reshape(n, c, h, w)


# speedup vs baseline: 1.3933x; 1.3933x over previous
import functools

import jax
import jax.numpy as jnp
from jax.experimental import pallas as pl
from jax.experimental.pallas import tpu as pltpu

_EPS = 1e-05
_GROUP = 64
_HB = 8


def _pass1_body(x_ref, w_ref, logits_ref, rsum_ref, rsumsq_ref, *, hb):
    k = pl.program_id(0)

    @pl.when(k == 0)
    def _():
        logits_ref[...] = jnp.zeros_like(logits_ref)
        rsum_ref[...] = jnp.zeros_like(rsum_ref)
        rsumsq_ref[...] = jnp.zeros_like(rsumsq_ref)

    xb = x_ref[...]
    xb16 = xb.astype(jnp.bfloat16)
    wb16 = w_ref[...].astype(jnp.bfloat16)
    acc = logits_ref[...]
    for i in range(hb):
        acc += jax.lax.dot_general(
            xb16[:, i, :], wb16[:, i, :],
            (((1,), (1,)), ((), ())), preferred_element_type=jnp.float32)
    logits_ref[...] = acc
    rsum_ref[...] += jnp.sum(jnp.sum(xb, axis=2), axis=1, keepdims=True)
    rsumsq_ref[...] += jnp.sum(jnp.sum(xb * xb, axis=2), axis=1, keepdims=True)


def _stats_body(logits_ref, b_ref, rsum_ref, rsumsq_ref, wrow_ref, brow_ref,
                scale_ref, shift_ref, *, d):
    logits = logits_ref[...] + b_ref[...]
    rows, group = logits.shape
    iota = jax.lax.broadcasted_iota(jnp.int32, (rows, group), 1)
    rowmax = jnp.max(logits, axis=1, keepdims=True)
    s = jnp.min(jnp.where(logits == rowmax, iota, group), axis=1, keepdims=True)
    onehot = (s == iota).astype(jnp.float32)
    cnt = jnp.sum(onehot, axis=0, keepdims=True)
    g_sum = jnp.sum(onehot * rsum_ref[...], axis=0, keepdims=True)
    g_sumsq = jnp.sum(onehot * rsumsq_ref[...], axis=0, keepdims=True)
    n_el = cnt * d
    mean = g_sum / jnp.maximum(n_el, 1.0)
    var = (g_sumsq - n_el * mean * mean) / jnp.maximum(n_el - 1.0, 1.0)
    m_r = jnp.sum(onehot * mean, axis=1, keepdims=True)
    v_r = jnp.sum(onehot * var, axis=1, keepdims=True)
    inv = wrow_ref[...] * jax.lax.rsqrt(v_r + _EPS)
    scale_ref[...] = inv
    shift_ref[...] = brow_ref[...] - m_r * inv


def _pass3_body(x_ref, scale_ref, shift_ref, out_ref):
    scale = scale_ref[...][:, :, None]
    shift = shift_ref[...][:, :, None]
    out_ref[...] = x_ref[...] * scale + shift


def kernel(x, fc_w, fc_b, weight, bias):
    n, c, h, w = x.shape
    d = h * w
    rows = n * c
    xv = x.reshape(rows, h, w)
    wv = fc_w.reshape(_GROUP, h, w)
    wrow = jnp.broadcast_to(weight.reshape(1, c), (n, c)).reshape(rows, 1)
    brow = jnp.broadcast_to(bias.reshape(1, c), (n, c)).reshape(rows, 1)

    hb = _HB
    hsteps = h // hb

    logits, rsum, rsumsq = pl.pallas_call(
        functools.partial(_pass1_body, hb=hb),
        grid=(hsteps,),
        in_specs=[
            pl.BlockSpec((rows, hb, w), lambda k: (0, k, 0)),
            pl.BlockSpec((_GROUP, hb, w), lambda k: (0, k, 0)),
        ],
        out_specs=[
            pl.BlockSpec((rows, _GROUP), lambda k: (0, 0)),
            pl.BlockSpec((rows, 1), lambda k: (0, 0)),
            pl.BlockSpec((rows, 1), lambda k: (0, 0)),
        ],
        out_shape=[
            jax.ShapeDtypeStruct((rows, _GROUP), jnp.float32),
            jax.ShapeDtypeStruct((rows, 1), jnp.float32),
            jax.ShapeDtypeStruct((rows, 1), jnp.float32),
        ],
        compiler_params=pltpu.CompilerParams(
            dimension_semantics=("arbitrary",)),
    )(xv, wv)

    scale, shift = pl.pallas_call(
        functools.partial(_stats_body, d=float(d)),
        in_specs=[
            pl.BlockSpec((rows, _GROUP), lambda: (0, 0)),
            pl.BlockSpec((1, _GROUP), lambda: (0, 0)),
            pl.BlockSpec((rows, 1), lambda: (0, 0)),
            pl.BlockSpec((rows, 1), lambda: (0, 0)),
            pl.BlockSpec((rows, 1), lambda: (0, 0)),
            pl.BlockSpec((rows, 1), lambda: (0, 0)),
        ],
        out_specs=[
            pl.BlockSpec((rows, 1), lambda: (0, 0)),
            pl.BlockSpec((rows, 1), lambda: (0, 0)),
        ],
        out_shape=[
            jax.ShapeDtypeStruct((rows, 1), jnp.float32),
            jax.ShapeDtypeStruct((rows, 1), jnp.float32),
        ],
    )(logits, fc_b.reshape(1, _GROUP), rsum, rsumsq, wrow, brow)

    out = pl.pallas_call(
        _pass3_body,
        grid=(hsteps,),
        in_specs=[
            pl.BlockSpec((rows, hb, w), lambda k: (0, k, 0)),
            pl.BlockSpec((rows, 1), lambda k: (0, 0)),
            pl.BlockSpec((rows, 1), lambda k: (0, 0)),
        ],
        out_specs=pl.BlockSpec((rows, hb, w), lambda k: (0, k, 0)),
        out_shape=jax.ShapeDtypeStruct((rows, h, w), jnp.float32),
        compiler_params=pltpu.CompilerParams(
            dimension_semantics=("arbitrary",)),
    )(xv, scale, shift)

    return out.reshape(n, c, h, w)


# speedup vs baseline: 4.6962x; 3.3706x over previous
import functools

import jax
import jax.numpy as jnp
from jax.experimental import pallas as pl
from jax.experimental.pallas import tpu as pltpu

_EPS = 1e-05
_GROUP = 64


def _kblk(d, limit=2048):
    best = 128
    for m in range(128, limit + 1, 128):
        if d % m == 0:
            best = m
    return best


def _pass1_body(x_ref, w_ref, logits_ref, rsum_ref, rsumsq_ref, *, nb):
    k = pl.program_id(0)

    @pl.when(k == 0)
    def _():
        logits_ref[...] = jnp.zeros_like(logits_ref)
        rsum_ref[...] = jnp.zeros_like(rsum_ref)
        rsumsq_ref[...] = jnp.zeros_like(rsumsq_ref)

    xb = x_ref[...]
    xb16 = xb.astype(jnp.bfloat16)
    wb16 = w_ref[...].astype(jnp.bfloat16)
    for i in range(nb):
        logits_ref[i] += jax.lax.dot_general(
            wb16, xb16[i],
            (((1,), (0,)), ((), ())), preferred_element_type=jnp.float32)
    rsum_ref[...] += jnp.sum(xb, axis=1)
    rsumsq_ref[...] += jnp.sum(xb * xb, axis=1)


def _stats_body(logits_ref, b_ref, rsum_ref, rsumsq_ref, wrow_ref, brow_ref,
                scale_ref, shift_ref, *, d):
    logits = logits_ref[...] + b_ref[...]
    nb, group, c = logits.shape
    iota = jax.lax.broadcasted_iota(jnp.int32, (nb, group, c), 1)
    gmax = jnp.max(logits, axis=1, keepdims=True)
    s = jnp.min(jnp.where(logits == gmax, iota, group), axis=1, keepdims=True)
    onehot = (s == iota).astype(jnp.float32)
    cnt = jnp.sum(jnp.sum(onehot, axis=2, keepdims=True), axis=0,
                  keepdims=True)
    rsumb = rsum_ref[...][:, None, :]
    rsumsqb = rsumsq_ref[...][:, None, :]
    g_sum = jnp.sum(jnp.sum(onehot * rsumb, axis=2, keepdims=True), axis=0,
                    keepdims=True)
    g_sumsq = jnp.sum(jnp.sum(onehot * rsumsqb, axis=2, keepdims=True), axis=0,
                      keepdims=True)
    n_el = cnt * d
    mean = g_sum / jnp.maximum(n_el, 1.0)
    var = (g_sumsq - n_el * mean * mean) / jnp.maximum(n_el - 1.0, 1.0)
    m_r = jnp.sum(onehot * mean, axis=1)
    v_r = jnp.sum(onehot * var, axis=1)
    inv = wrow_ref[...] * jax.lax.rsqrt(v_r + _EPS)
    scale_ref[...] = inv
    shift_ref[...] = brow_ref[...] - m_r * inv


def _pass3_body(x_ref, scale_ref, shift_ref, out_ref):
    scale = scale_ref[...][:, None, :]
    shift = shift_ref[...][:, None, :]
    out_ref[...] = x_ref[...] * scale + shift


def kernel(x, fc_w, fc_b, weight, bias):
    n, c, h, w = x.shape
    d = h * w
    xm = x.transpose(0, 2, 3, 1).reshape(n, d, c)
    wrow = jnp.broadcast_to(weight.reshape(1, c), (n, c))
    brow = jnp.broadcast_to(bias.reshape(1, c), (n, c))

    kblk = _kblk(d)
    ksteps = d // kblk

    logits, rsum, rsumsq = pl.pallas_call(
        functools.partial(_pass1_body, nb=n),
        grid=(ksteps,),
        in_specs=[
            pl.BlockSpec((n, kblk, c), lambda k: (0, k, 0)),
            pl.BlockSpec((_GROUP, kblk), lambda k: (0, k)),
        ],
        out_specs=[
            pl.BlockSpec((n, _GROUP, c), lambda k: (0, 0, 0)),
            pl.BlockSpec((n, c), lambda k: (0, 0)),
            pl.BlockSpec((n, c), lambda k: (0, 0)),
        ],
        out_shape=[
            jax.ShapeDtypeStruct((n, _GROUP, c), jnp.float32),
            jax.ShapeDtypeStruct((n, c), jnp.float32),
            jax.ShapeDtypeStruct((n, c), jnp.float32),
        ],
        compiler_params=pltpu.CompilerParams(
            dimension_semantics=("arbitrary",)),
    )(xm, fc_w)

    scale, shift = pl.pallas_call(
        functools.partial(_stats_body, d=float(d)),
        in_specs=[
            pl.BlockSpec((n, _GROUP, c), lambda: (0, 0, 0)),
            pl.BlockSpec((1, _GROUP, 1), lambda: (0, 0, 0)),
            pl.BlockSpec((n, c), lambda: (0, 0)),
            pl.BlockSpec((n, c), lambda: (0, 0)),
            pl.BlockSpec((n, c), lambda: (0, 0)),
            pl.BlockSpec((n, c), lambda: (0, 0)),
        ],
        out_specs=[
            pl.BlockSpec((n, c), lambda: (0, 0)),
            pl.BlockSpec((n, c), lambda: (0, 0)),
        ],
        out_shape=[
            jax.ShapeDtypeStruct((n, c), jnp.float32),
            jax.ShapeDtypeStruct((n, c), jnp.float32),
        ],
    )(logits, fc_b.reshape(1, _GROUP, 1), rsum, rsumsq, wrow, brow)

    out = pl.pallas_call(
        _pass3_body,
        grid=(ksteps,),
        in_specs=[
            pl.BlockSpec((n, kblk, c), lambda k: (0, k, 0)),
            pl.BlockSpec((n, c), lambda k: (0, 0)),
            pl.BlockSpec((n, c), lambda k: (0, 0)),
        ],
        out_specs=pl.BlockSpec((n, kblk, c), lambda k: (0, k, 0)),
        out_shape=jax.ShapeDtypeStruct((n, d, c), jnp.float32),
        compiler_params=pltpu.CompilerParams(
            dimension_semantics=("arbitrary",)),
    )(xm, scale, shift)

    return out.reshape(n, h, w, c).transpose(0, 3, 1, 2)


# speedup vs baseline: 4.7957x; 1.0212x over previous
import functools

import jax
import jax.numpy as jnp
from jax.experimental import pallas as pl
from jax.experimental.pallas import tpu as pltpu

_EPS = 1e-05
_GROUP = 64


def _kblk(d, limit=2048):
    best = 128
    for m in range(128, limit + 1, 128):
        if d % m == 0:
            best = m
    return best


def _fused_body(x_ref, w_ref, b_ref, wrow_ref, brow_ref, out_ref,
                logits_s, rsum_s, rsumsq_s, scale_s, shift_s, *, nb, d):
    p = pl.program_id(0)
    k = pl.program_id(1)

    @pl.when((p == 0) & (k == 0))
    def _():
        logits_s[...] = jnp.zeros_like(logits_s)
        rsum_s[...] = jnp.zeros_like(rsum_s)
        rsumsq_s[...] = jnp.zeros_like(rsumsq_s)

    @pl.when(p == 0)
    def _():
        xb = x_ref[...]
        xb16 = xb.astype(jnp.bfloat16)
        wb16 = w_ref[...].astype(jnp.bfloat16)
        for i in range(nb):
            logits_s[i] += jax.lax.dot_general(
                wb16, xb16[i],
                (((1,), (0,)), ((), ())), preferred_element_type=jnp.float32)
        rsum_s[...] += jnp.sum(xb, axis=1)
        rsumsq_s[...] += jnp.sum(xb * xb, axis=1)

    @pl.when((p == 1) & (k == 0))
    def _():
        logits = logits_s[...] + b_ref[...]
        _, group, c = logits.shape
        iota = jax.lax.broadcasted_iota(jnp.int32, (nb, group, c), 1)
        gmax = jnp.max(logits, axis=1, keepdims=True)
        s = jnp.min(jnp.where(logits == gmax, iota, group), axis=1,
                    keepdims=True)
        onehot = (s == iota).astype(jnp.float32)
        cnt = jnp.sum(jnp.sum(onehot, axis=2, keepdims=True), axis=0,
                      keepdims=True)
        rsumb = rsum_s[...][:, None, :]
        rsumsqb = rsumsq_s[...][:, None, :]
        g_sum = jnp.sum(jnp.sum(onehot * rsumb, axis=2, keepdims=True),
                        axis=0, keepdims=True)
        g_sumsq = jnp.sum(jnp.sum(onehot * rsumsqb, axis=2, keepdims=True),
                          axis=0, keepdims=True)
        n_el = cnt * d
        mean = g_sum / jnp.maximum(n_el, 1.0)
        var = (g_sumsq - n_el * mean * mean) / jnp.maximum(n_el - 1.0, 1.0)
        m_r = jnp.sum(onehot * mean, axis=1)
        v_r = jnp.sum(onehot * var, axis=1)
        inv = wrow_ref[...] * jax.lax.rsqrt(v_r + _EPS)
        scale_s[...] = inv
        shift_s[...] = brow_ref[...] - m_r * inv

    @pl.when(p == 1)
    def _():
        scale = scale_s[...][:, None, :]
        shift = shift_s[...][:, None, :]
        out_ref[...] = x_ref[...] * scale + shift


def kernel(x, fc_w, fc_b, weight, bias):
    n, c, h, w = x.shape
    d = h * w
    xm = x.transpose(0, 2, 3, 1).reshape(n, d, c)
    wrow = jnp.broadcast_to(weight.reshape(1, c), (n, c))
    brow = jnp.broadcast_to(bias.reshape(1, c), (n, c))

    kblk = _kblk(d)
    ksteps = d // kblk

    out = pl.pallas_call(
        functools.partial(_fused_body, nb=n, d=float(d)),
        grid=(2, ksteps),
        in_specs=[
            pl.BlockSpec((n, kblk, c), lambda p, k: (0, k, 0)),
            pl.BlockSpec((_GROUP, kblk), lambda p, k: (0, k * (1 - p))),
            pl.BlockSpec((1, _GROUP, 1), lambda p, k: (0, 0, 0)),
            pl.BlockSpec((n, c), lambda p, k: (0, 0)),
            pl.BlockSpec((n, c), lambda p, k: (0, 0)),
        ],
        out_specs=pl.BlockSpec((n, kblk, c), lambda p, k: (0, k * p, 0)),
        out_shape=jax.ShapeDtypeStruct((n, d, c), jnp.float32),
        scratch_shapes=[
            pltpu.VMEM((n, _GROUP, c), jnp.float32),
            pltpu.VMEM((n, c), jnp.float32),
            pltpu.VMEM((n, c), jnp.float32),
            pltpu.VMEM((n, c), jnp.float32),
            pltpu.VMEM((n, c), jnp.float32),
        ],
        compiler_params=pltpu.CompilerParams(
            dimension_semantics=("arbitrary", "arbitrary")),
    )(xm, fc_w, fc_b.reshape(1, _GROUP, 1), wrow, brow)

    return out.reshape(n, h, w, c).transpose(0, 3, 1, 2)


# speedup vs baseline: 5.0003x; 1.0427x over previous
import functools

import jax
import jax.numpy as jnp
from jax.experimental import pallas as pl
from jax.experimental.pallas import tpu as pltpu

_EPS = 1e-05
_GROUP = 64


def _kblk(d, limit=4096):
    best = 128
    for m in range(128, limit + 1, 128):
        if d % m == 0:
            best = m
    return best


def _fused_body(x_ref, w_ref, b_ref, wrow_ref, brow_ref, out_ref,
                logits_s, rsum_s, rsumsq_s, scale_s, shift_s, *, nb, d):
    p = pl.program_id(0)
    k = pl.program_id(1)

    @pl.when((p == 0) & (k == 0))
    def _():
        logits_s[...] = jnp.zeros_like(logits_s)
        rsum_s[...] = jnp.zeros_like(rsum_s)
        rsumsq_s[...] = jnp.zeros_like(rsumsq_s)

    @pl.when(p == 0)
    def _():
        xb = x_ref[...]
        xb16 = xb.astype(jnp.bfloat16)
        wb16 = w_ref[...].astype(jnp.bfloat16)
        for i in range(nb):
            logits_s[i] += jax.lax.dot_general(
                wb16, xb16[i],
                (((1,), (0,)), ((), ())), preferred_element_type=jnp.float32)
        rsum_s[...] += jnp.sum(xb, axis=1)
        rsumsq_s[...] += jnp.sum(xb * xb, axis=1)

    @pl.when((p == 1) & (k == 0))
    def _():
        logits = logits_s[...] + b_ref[...]
        _, group, c = logits.shape
        iota = jax.lax.broadcasted_iota(jnp.int32, (nb, group, c), 1)
        gmax = jnp.max(logits, axis=1, keepdims=True)
        s = jnp.min(jnp.where(logits == gmax, iota, group), axis=1,
                    keepdims=True)
        onehot = (s == iota).astype(jnp.float32)
        cnt = jnp.sum(jnp.sum(onehot, axis=2, keepdims=True), axis=0,
                      keepdims=True)
        rsumb = rsum_s[...][:, None, :]
        rsumsqb = rsumsq_s[...][:, None, :]
        g_sum = jnp.sum(jnp.sum(onehot * rsumb, axis=2, keepdims=True),
                        axis=0, keepdims=True)
        g_sumsq = jnp.sum(jnp.sum(onehot * rsumsqb, axis=2, keepdims=True),
                          axis=0, keepdims=True)
        n_el = cnt * d
        mean = g_sum / jnp.maximum(n_el, 1.0)
        var = (g_sumsq - n_el * mean * mean) / jnp.maximum(n_el - 1.0, 1.0)
        m_r = jnp.sum(onehot * mean, axis=1)
        v_r = jnp.sum(onehot * var, axis=1)
        inv = wrow_ref[...] * jax.lax.rsqrt(v_r + _EPS)
        scale_s[...] = inv
        shift_s[...] = brow_ref[...] - m_r * inv

    @pl.when(p == 1)
    def _():
        scale = scale_s[...][:, None, :]
        shift = shift_s[...][:, None, :]
        out_ref[...] = x_ref[...] * scale + shift


def kernel(x, fc_w, fc_b, weight, bias):
    n, c, h, w = x.shape
    d = h * w
    xm = x.transpose(0, 2, 3, 1).reshape(n, d, c)
    wrow = jnp.broadcast_to(weight.reshape(1, c), (n, c))
    brow = jnp.broadcast_to(bias.reshape(1, c), (n, c))

    kblk = _kblk(d)
    ksteps = d // kblk

    out = pl.pallas_call(
        functools.partial(_fused_body, nb=n, d=float(d)),
        grid=(2, ksteps),
        in_specs=[
            pl.BlockSpec((n, kblk, c), lambda p, k: (0, k, 0)),
            pl.BlockSpec((_GROUP, kblk), lambda p, k: (0, k * (1 - p))),
            pl.BlockSpec((1, _GROUP, 1), lambda p, k: (0, 0, 0)),
            pl.BlockSpec((n, c), lambda p, k: (0, 0)),
            pl.BlockSpec((n, c), lambda p, k: (0, 0)),
        ],
        out_specs=pl.BlockSpec((n, kblk, c), lambda p, k: (0, k * p, 0)),
        out_shape=jax.ShapeDtypeStruct((n, d, c), jnp.float32),
        scratch_shapes=[
            pltpu.VMEM((n, _GROUP, c), jnp.float32),
            pltpu.VMEM((n, c), jnp.float32),
            pltpu.VMEM((n, c), jnp.float32),
            pltpu.VMEM((n, c), jnp.float32),
            pltpu.VMEM((n, c), jnp.float32),
        ],
        compiler_params=pltpu.CompilerParams(
            dimension_semantics=("arbitrary", "arbitrary"),
            vmem_limit_bytes=60 * 1024 * 1024),
    )(xm, fc_w, fc_b.reshape(1, _GROUP, 1), wrow, brow)

    return out.reshape(n, h, w, c).transpose(0, 3, 1, 2)
